# baseline (speedup 1.0000x reference)
import functools

import jax
import jax.numpy as jnp
from jax import lax
from jax.experimental import pallas as pl
from jax.experimental.pallas import tpu as pltpu
from jax.experimental.pallas import tpu_sc as plsc

_N = 16384
_K = int(_N * 0.1)
_NSUB = 16
_E = _N // _NSUB
_NV = _E // 16
_CROW = _E + 16
_INT_MIN = -(2 ** 31)


def _body(loss_hbm, out_hbm, xv, mk, lh, lhr, gridv, ckey, candg, ck2,
          outv, grid_s, cand_s):
    c = lax.axis_index("c")
    wid = lax.axis_index("s")
    on0 = c == 0
    li = lax.iota(jnp.int32, 16)
    z16i = jnp.zeros((16,), jnp.int32)
    z16f = jnp.zeros((16,), jnp.float32)
    ones16 = jnp.ones((16,), jnp.int32)

    def _trips(n):
        return jnp.where(on0, n, 0)

    def _scal(v, lane):
        return jnp.sum(jnp.where(li == lane, v, jnp.zeros_like(v)))

    with jax.named_scope("p1_load"):
        pltpu.sync_copy(loss_hbm.at[pl.ds(wid * _E, _E)], xv)

    def _keys(i, carry):
        for u in range(4):
            x = xv[pl.ds(i * 64 + u * 16, 16)]
            b = lax.bitcast_convert_type(x, jnp.int32)
            mk[pl.ds(i * 64 + u * 16, 16)] = jnp.where(b < 0, ~b, b ^ _INT_MIN)
        return carry

    with jax.named_scope("c1_keys"):
        lax.fori_loop(0, _trips(_NV // 4), _keys, 0)

    def _zero(i, carry):
        for u in range(8):
            lh[pl.ds(i * 128 + u * 16, 16)] = z16i
        return carry

    with jax.named_scope("c2_zero"):
        lax.fori_loop(0, _trips(32), _zero, 0)

    lidx = li * 256
    sh24 = jnp.full((16,), 24, jnp.int32)

    def _hist(i, carry):
        for u in range(4):
            m = mk[pl.ds(i * 64 + u * 16, 16)]
            binv = lax.shift_right_logical(m, sh24)
            plsc.addupdate_scatter(lh, [lidx + binv], ones16)
        return carry

    with jax.named_scope("c3_hist"):
        lax.fori_loop(0, _trips(_NV // 4), _hist, 0)

    def _red(i, carry):
        a = z16i
        for j in range(16):
            a = a + lh[pl.ds(j * 256 + i * 16, 16)]
        lhr[pl.ds(i * 16, 16)] = a
        return carry

    with jax.named_scope("c4_red"):
        lax.fori_loop(0, _trips(16), _red, 0)
    with jax.named_scope("p2_gridw"):
        pltpu.sync_copy(lhr, grid_s.at[pl.ds(wid * 256, 256)])
    with jax.named_scope("p3_bar1"):
        plsc.subcore_barrier()

    with jax.named_scope("p4_gridr"):
        pltpu.sync_copy(grid_s, gridv)

    def _merge(i, carry):
        cum, bstar, cbelow = carry
        g = z16i
        for j in range(16):
            g = g + gridv[pl.ds(j * 256 + i * 16, 16)]
        cs = plsc.cumsum(g)
        tot = jnp.sum(g)
        msk = (cum + cs) >= _K
        f = jnp.max(plsc.all_reduce_ffs(msk))
        found = jnp.logical_and(bstar < 0, f < 16)
        below = jnp.sum(jnp.where(li < f, g, 0))
        bstar = jnp.where(found, i * 16 + f, bstar)
        cbelow = jnp.where(found, cum + below, cbelow)
        return (cum + tot, bstar, cbelow)

    with jax.named_scope("c5_merge"):
        _, bstar, cbelow = lax.fori_loop(
            0, _trips(16), _merge, (jnp.int32(0), jnp.int32(-1), jnp.int32(0)))
    rk1 = jnp.int32(_K) - cbelow

    def _compact(i, carry):
        off_v, s_lo, z_lo = carry
        m = mk[pl.ds(i * 16, 16)]
        x = xv[pl.ds(i * 16, 16)]
        top8 = lax.shift_right_logical(m, sh24)
        is_c = top8 == bstar
        is_lo = top8 < bstar
        idx = off_v + plsc.cumsum(jnp.where(is_c, 1, 0)) - 1
        plsc.store_scatter(ckey, [idx], m, mask=is_c)
        off_v = off_v + plsc.all_reduce_population_count(is_c)
        s_lo = s_lo + jnp.where(is_lo, x, z16f)
        z_lo = z_lo + jnp.where(
            jnp.logical_and(is_lo, x == jnp.float32(0)), 1, 0)
        return (off_v, s_lo, z_lo)

    with jax.named_scope("c6_compact"):
        off_v, s_lo_v, z_lo_v = lax.fori_loop(
            0, _trips(_NV), _compact, (z16i, z16f, z16i))
    cnt_c = jnp.max(off_v)
    s_lo = jnp.sum(s_lo_v)
    z_lo = jnp.sum(z_lo_v)
    ckey[pl.ds(_E, 16)] = jnp.where(
        li == 0, cnt_c,
        jnp.where(li == 1, lax.bitcast_convert_type(s_lo, jnp.int32),
                  jnp.where(li == 2, z_lo, 0)))
    with jax.named_scope("p5_candw"):
        pltpu.sync_copy(ckey, cand_s.at[pl.ds(wid * _CROW, _CROW)])
    with jax.named_scope("p6_bar2"):
        plsc.subcore_barrier()

    @pl.when(jnp.logical_and(wid == 0, on0))
    def _():
        with jax.named_scope("p7_candr"):
            pltpu.sync_copy(cand_s, candg)
        s_g = jnp.float32(0)
        z_g = jnp.int32(0)
        c_all = z16i
        for t in range(16):
            meta = candg[pl.ds(t * _CROW + _E, 16)]
            cnt_t = _scal(meta, 0)
            s_g = s_g + lax.bitcast_convert_type(_scal(meta, 1), jnp.float32)
            z_g = z_g + _scal(meta, 2)

            def _gather(j, off_v):
                v = candg[pl.ds(t * _CROW + j * 16, 16)]
                valid = (j * 16 + li) < cnt_t
                idx = off_v + plsc.cumsum(jnp.where(valid, 1, 0)) - 1
                plsc.store_scatter(ck2, [idx], v, mask=valid)
                return off_v + plsc.all_reduce_population_count(valid)

            c_all = lax.fori_loop(0, (cnt_t + 15) >> 4, _gather, c_all)

        c_sc = jnp.max(c_all)
        base = lax.shift_left(bstar, 24)
        nvc = (c_sc + 15) >> 4

        def _bit(i, p):
            t_try = p | lax.shift_left(jnp.int32(1), 23 - i)
            st = t_try ^ _INT_MIN
            cv = z16i
            for j in range(8):
                v = ck2[pl.ds(j * 16, 16)]
                sm = v ^ _INT_MIN
                msk = jnp.logical_and(sm < st, (j * 16 + li) < c_sc)
                cv = cv + jnp.where(msk, 1, 0)

            def _cnt(j, acc):
                v = ck2[pl.ds(j * 16, 16)]
                sm = v ^ _INT_MIN
                msk = jnp.logical_and(sm < st, (j * 16 + li) < c_sc)
                return acc + jnp.where(msk, 1, 0)

            cc = jnp.sum(lax.fori_loop(8, jnp.maximum(nvc, 8), _cnt, cv))
            return jnp.where(cc < rk1, t_try, p)

        with jax.named_scope("c7_bsearch"):
            p_key = lax.fori_loop(0, 24, _bit, base)
        sp = p_key ^ _INT_MIN

        def _cstat(j, carry):
            s_c, z_c, n_lt = carry
            v = ck2[pl.ds(j * 16, 16)]
            sm = v ^ _INT_MIN
            sel = jnp.logical_and(sm < sp, (j * 16 + li) < c_sc)
            cv = lax.bitcast_convert_type(
                jnp.where(v < 0, v ^ _INT_MIN, ~v), jnp.float32)
            s_c = s_c + jnp.where(sel, cv, z16f)
            z_c = z_c + jnp.where(
                jnp.logical_and(sel, cv == jnp.float32(0)), 1, 0)
            n_lt = n_lt + jnp.where(sel, 1, 0)
            return (s_c, z_c, n_lt)

        s_c_v, z_c_v, n_lt_v = lax.fori_loop(
            0, nvc, _cstat, (z16f, z16i, z16i))
        rk_fin = (rk1 - jnp.sum(n_lt_v)).astype(jnp.float32)

        pv = jnp.full((16,), 1, jnp.int32) * p_key
        t_v = lax.bitcast_convert_type(
            jnp.where(pv < 0, pv ^ _INT_MIN, ~pv), jnp.float32)
        total = s_g + jnp.sum(s_c_v) + t_v * rk_fin
        denom = (jnp.float32(_K) - (z_g + jnp.sum(z_c_v)).astype(jnp.float32)
                 - jnp.where(t_v == jnp.float32(0), rk_fin, 0.0))
        outv[...] = total / denom
        pltpu.sync_copy(outv, out_hbm)


@functools.partial(
    pl.kernel,
    out_type=jax.ShapeDtypeStruct((16,), jnp.float32),
    mesh=plsc.VectorSubcoreMesh(
        core_axis_name="c", subcore_axis_name="s",
        num_cores=2, num_subcores=_NSUB),
    compiler_params=pltpu.CompilerParams(
        needs_layout_passes=False, skip_device_barrier=True),
    scratch_types=[
        pltpu.VMEM((_E,), jnp.float32),
        pltpu.VMEM((_E,), jnp.int32),
        pltpu.VMEM((16 * 256,), jnp.int32),
        pltpu.VMEM((256,), jnp.int32),
        pltpu.VMEM((16 * 256,), jnp.int32),
        pltpu.VMEM((_CROW,), jnp.int32),
        pltpu.VMEM((16 * _CROW,), jnp.int32),
        pltpu.VMEM((_N + 16,), jnp.int32),
        pltpu.VMEM((16,), jnp.float32),
        pltpu.VMEM_SHARED((16 * 256,), jnp.int32),
        pltpu.VMEM_SHARED((16 * _CROW,), jnp.int32),
    ],
)
def _select_mean(loss_hbm, out_hbm, *scratch):
    _body(loss_hbm, out_hbm, *scratch)


def kernel(loss, gradients):
    del gradients
    return _select_mean(loss)[0]


# speedup vs baseline: 1.0795x; 1.0795x over previous
import functools

import jax
import jax.numpy as jnp
from jax import lax
from jax.experimental import pallas as pl
from jax.experimental.pallas import tpu as pltpu
from jax.experimental.pallas import tpu_sc as plsc

_N = 16384
_K = int(_N * 0.1)
_NSUB = 16
_E = _N // _NSUB
_NV = _E // 16
_INT_MIN = -(2 ** 31)


def _body(loss_hbm, out_hbm, xv, mk, lh, lhr, gridv, decb, accrow, accg,
          outv, grid_s, dec_s, acc_s):
    c = lax.axis_index("c")
    wid = lax.axis_index("s")
    on0 = c == 0
    li = lax.iota(jnp.int32, 16)
    z16i = jnp.zeros((16,), jnp.int32)
    ones16 = jnp.ones((16,), jnp.int32)

    def _trips(n):
        return jnp.where(on0, n, 0)

    pltpu.sync_copy(loss_hbm.at[pl.ds(wid * _E, _E)], xv)

    def _keys(i, carry):
        x = xv[pl.ds(i * 16, 16)]
        b = lax.bitcast_convert_type(x, jnp.int32)
        mk[pl.ds(i * 16, 16)] = jnp.where(b < 0, ~b, b ^ _INT_MIN)
        return carry

    lax.fori_loop(0, _trips(_NV), _keys, 0)

    prefix = jnp.int32(0)
    rk = jnp.int32(_K)

    for r in range(4):
        shift = 24 - 8 * r

        def _zero(i, carry):
            lh[pl.ds(i * 16, 16)] = z16i
            return carry

        lax.fori_loop(0, _trips(256), _zero, 0)

        lidx = li * 256
        shv = jnp.full((16,), shift, jnp.int32)
        shv8 = jnp.full((16,), shift + 8, jnp.int32)

        def _hist(i, carry):
            m = mk[pl.ds(i * 16, 16)]
            binv = lax.shift_right_logical(m, shv) & 255
            if r == 0:
                plsc.addupdate_scatter(lh, [lidx + binv], ones16)
            else:
                inr = lax.shift_right_logical(m, shv8) == prefix
                plsc.addupdate_scatter(lh, [lidx + binv], ones16, mask=inr)
            return carry

        lax.fori_loop(0, _trips(_NV), _hist, 0)

        def _red(i, carry):
            a = z16i
            for j in range(16):
                a = a + lh[pl.ds(j * 256 + i * 16, 16)]
            lhr[pl.ds(i * 16, 16)] = a
            return carry

        lax.fori_loop(0, _trips(16), _red, 0)
        pltpu.sync_copy(lhr, grid_s.at[r, wid])
        plsc.subcore_barrier()

        @pl.when(jnp.logical_and(wid == 0, on0))
        def _():
            pltpu.sync_copy(grid_s.at[r], gridv)

            def _merge(i, carry):
                cum, bstar, cbelow = carry
                g = z16i
                for j in range(16):
                    g = g + gridv[j, pl.ds(i * 16, 16)]
                cs = plsc.cumsum(g)
                tot = jnp.sum(g)
                msk = (cum + cs) >= rk
                f = jnp.max(plsc.all_reduce_ffs(msk))
                found = jnp.logical_and(bstar < 0, f < 16)
                below = jnp.sum(jnp.where(li < f, g, 0))
                bstar = jnp.where(found, i * 16 + f, bstar)
                cbelow = jnp.where(found, cum + below, cbelow)
                return (cum + tot, bstar, cbelow)

            _, bstar, cbelow = lax.fori_loop(
                0, 16, _merge, (jnp.int32(0), jnp.int32(-1), jnp.int32(0)))
            decb[...] = jnp.where(
                li == 0, prefix * 256 + bstar,
                jnp.where(li == 1, rk - cbelow, 0))
            pltpu.sync_copy(decb, dec_s.at[r])

        plsc.subcore_barrier()
        pltpu.sync_copy(dec_s.at[r], decb)
        d = decb[...]
        prefix = jnp.sum(jnp.where(li == 0, d, 0))
        rk = jnp.sum(jnp.where(li == 1, d, 0))

    sP = prefix ^ _INT_MIN

    def _final(i, carry):
        s_acc, z_acc = carry
        m = mk[pl.ds(i * 16, 16)]
        x = xv[pl.ds(i * 16, 16)]
        sel = (m ^ _INT_MIN) < sP
        s_acc = s_acc + jnp.where(sel, x, jnp.float32(0))
        z_acc = z_acc + jnp.where(
            jnp.logical_and(sel, x == jnp.float32(0)), 1, 0)
        return (s_acc, z_acc)

    s_acc, z_acc = lax.fori_loop(
        0, _trips(_NV), _final, (jnp.zeros((16,), jnp.float32), z16i))
    s_l = jnp.sum(s_acc)
    z_l = jnp.sum(z_acc).astype(jnp.float32)
    accrow[...] = jnp.where(li == 0, s_l, jnp.where(li == 1, z_l, 0.0))
    pltpu.sync_copy(accrow, acc_s.at[wid])
    plsc.subcore_barrier()

    @pl.when(jnp.logical_and(wid == 0, on0))
    def _():
        pltpu.sync_copy(acc_s, accg)
        tv = jnp.zeros((16,), jnp.float32)
        for j in range(16):
            tv = tv + accg[j, pl.ds(0, 16)]
        s_g = jnp.sum(jnp.where(li == 0, tv, 0.0))
        z_g = jnp.sum(jnp.where(li == 1, tv, 0.0))
        pv = jnp.full((16,), 1, jnp.int32) * prefix
        bv = jnp.where(pv < 0, pv ^ _INT_MIN, ~pv)
        t_v = lax.bitcast_convert_type(bv, jnp.float32)
        rkf = rk.astype(jnp.float32)
        total = s_g + t_v * rkf
        denom = (jnp.float32(_K) - z_g
                 - jnp.where(t_v == jnp.float32(0), rkf, 0.0))
        outv[...] = total / denom
        pltpu.sync_copy(outv, out_hbm)


@functools.partial(
    pl.kernel,
    out_type=jax.ShapeDtypeStruct((16,), jnp.float32),
    mesh=plsc.VectorSubcoreMesh(
        core_axis_name="c", subcore_axis_name="s",
        num_cores=2, num_subcores=_NSUB),
    compiler_params=pltpu.CompilerParams(
        needs_layout_passes=False, skip_device_barrier=True),
    scratch_types=[
        pltpu.VMEM((_E,), jnp.float32),
        pltpu.VMEM((_E,), jnp.int32),
        pltpu.VMEM((16 * 256,), jnp.int32),
        pltpu.VMEM((256,), jnp.int32),
        pltpu.VMEM((16, 256), jnp.int32),
        pltpu.VMEM((16,), jnp.int32),
        pltpu.VMEM((16,), jnp.float32),
        pltpu.VMEM((16, 16), jnp.float32),
        pltpu.VMEM((16,), jnp.float32),
        pltpu.VMEM_SHARED((4, 16, 256), jnp.int32),
        pltpu.VMEM_SHARED((4, 16), jnp.int32),
        pltpu.VMEM_SHARED((16, 16), jnp.float32),
    ],
)
def _select_mean(loss_hbm, out_hbm, *scratch):
    _body(loss_hbm, out_hbm, *scratch)


def kernel(loss, gradients):
    del gradients
    return _select_mean(loss)[0]
